# Initial kernel scaffold; baseline (speedup 1.0000x reference)
import jax
import jax.numpy as jnp
from jax.experimental import pallas as pl


def kernel(charge_coefficients, positions, volumes, batch):
    raise NotImplementedError("write your pallas kernel here")


# speedup vs baseline: 1.6585x; 1.6585x over previous
import functools
import math

import jax
import jax.numpy as jnp
from jax import lax
from jax.experimental import pallas as pl
from jax.experimental.pallas import tpu as pltpu
from jax.experimental.pallas import tpu_sc as plsc

_FIELD_CONSTANT = 4.0 * math.pi * 14.399645351950548
_CUBIC_MADELUNG = -2.8372974794806
_CONST = _FIELD_CONSTANT / (4.0 * math.pi)
_PI = math.pi

_N = 1600000
_B = 4096
_NC = 2
_NS = 16
_NW = _NC * _NS
_W = _N // _NW
_CH = 2000
_NCHUNK = _W // _CH
_NVEC = _CH // 16
_ACC = 5 * _B


def _sc_segment_sums(cc_flat, pos_flat, batch):
    mesh = plsc.VectorSubcoreMesh(core_axis_name="c", subcore_axis_name="s")

    @functools.partial(
        pl.kernel,
        out_type=jax.ShapeDtypeStruct((_NW, _ACC), jnp.float32),
        mesh=mesh,
        scratch_types=[
            pltpu.VMEM((_ACC,), jnp.float32),
            pltpu.VMEM((_CH * 4,), jnp.float32),
            pltpu.VMEM((_CH * 3,), jnp.float32),
            pltpu.VMEM((_CH,), jnp.int32),
        ],
        compiler_params=pltpu.CompilerParams(needs_layout_passes=False),
    )
    def body(cc_hbm, pos_hbm, b_hbm, out_hbm, acc, ccb, posb, bb):
        c = lax.axis_index("c")
        s = lax.axis_index("s")
        wid = s * _NC + c
        base_atom = wid * _W
        iota = lax.iota(jnp.int32, 16)
        zeros16f = jnp.zeros((16,), jnp.float32)

        def zero_chunk(i, carry):
            acc[pl.ds(i * 16, 16)] = zeros16f
            return carry
        lax.fori_loop(0, _ACC // 16, zero_chunk, 0)

        def vec_body(vi, carry):
            b0 = vi * 16
            rows = b0 + iota
            ids = bb[pl.ds(b0, 16)]
            r4 = rows * 4
            r3 = rows * 3
            q = plsc.load_gather(ccb, [r4])
            c1 = plsc.load_gather(ccb, [r4 + 1])
            c2 = plsc.load_gather(ccb, [r4 + 2])
            c3 = plsc.load_gather(ccb, [r4 + 3])
            px = plsc.load_gather(posb, [r3])
            py = plsc.load_gather(posb, [r3 + 1])
            pz = plsc.load_gather(posb, [r3 + 2])

            d0 = q * px + c3
            d1 = q * py + c1
            d2 = q * pz + c2
            r2 = px * px + py * py + pz * pz
            pdr = px * c3 + py * c1 + pz * c2
            qq = r2 * q + 2.0 * pdr

            nxt = b0 + jnp.minimum(iota + 1, 15)
            ids_n = plsc.load_gather(bb, [nxt])
            is_bound = ids != ids_n
            is_last = jnp.logical_or(is_bound, iota == 15)

            for k, contrib in enumerate((q, d0, d1, d2, qq)):
                cum = plsc.cumsum(contrib)
                off = jnp.int32(k * _B)
                plsc.addupdate_scatter(acc, [ids + off], cum, mask=is_last)
                plsc.addupdate_scatter(acc, [ids_n + off], -cum,
                                       mask=is_bound)
            return carry

        def chunk_body(ci, carry):
            start = base_atom + ci * _CH
            pltpu.sync_copy(cc_hbm.at[pl.ds(start * 4, _CH * 4)], ccb)
            pltpu.sync_copy(pos_hbm.at[pl.ds(start * 3, _CH * 3)], posb)
            pltpu.sync_copy(b_hbm.at[pl.ds(start, _CH)], bb)
            return lax.fori_loop(0, _NVEC, vec_body, carry)

        lax.fori_loop(0, _NCHUNK, chunk_body, 0)
        pltpu.sync_copy(acc, out_hbm.at[wid])

    return body(cc_flat, pos_flat, batch)


def _tc_combine_body(p_ref, v_ref, o_ref):
    p = jnp.sum(p_ref[...], axis=0)
    nb = _B // 128
    t = p[0 * nb:1 * nb]
    d0 = p[1 * nb:2 * nb]
    d1 = p[2 * nb:3 * nb]
    d2 = p[3 * nb:4 * nb]
    quad = p[4 * nb:5 * nb]
    vol = v_ref[...]
    ls = jnp.power(vol, 0.3333)
    de = 0.5 * _CUBIC_MADELUNG * _CONST * t * t / ls
    de = de + 2.0 * _CONST * _PI * (d0 * d0 + d1 * d1 + d2 * d2) / (3.0 * vol)
    de = de - 2.0 * _CONST * _PI * t * quad / (3.0 * vol)
    o_ref[...] = de


def kernel(charge_coefficients, positions, volumes, batch):
    batch_i = batch.astype(jnp.int32)
    partials = _sc_segment_sums(
        charge_coefficients.reshape(_N * 4),
        positions.reshape(_N * 3),
        batch_i,
    )
    vol2 = volumes.reshape(_B // 128, 128)
    de = pl.pallas_call(
        _tc_combine_body,
        out_shape=jax.ShapeDtypeStruct((_B // 128, 128), jnp.float32),
    )(partials.reshape(_NW, 5 * _B // 128, 128), vol2)
    return de.reshape(_B)
